# baseline (speedup 1.0000x reference)
import jax
import jax.numpy as jnp
from jax import lax
from jax.experimental import pallas as pl
from jax.experimental.pallas import tpu as pltpu
from jax.experimental.pallas import tpu_sc as plsc

VOCAB = 1000000
EMB_DIM = 64
NC = 2
NS = 16
NW = NC * NS

B_I = 16384
B_J = 50
LANE = 128
N_IT = B_I // LANE
IT_PER_W = N_IT // NW
UNITS = IT_PER_W * B_J


def _emb_body(x_hbm, tpk_hbm, out_hbm, ibuf, sidx, par, gb0, gb1, rb0, rb1,
              gsem0, gsem1, ssem0, ssem1):
    wid = lax.axis_index("s") * NC + lax.axis_index("c")
    iota = lax.iota(jnp.int32, 16)

    def it_body(itl, carry):
        it = wid * IT_PER_W + itl
        col = it * LANE

        def jt_body(jt, c2):
            pltpu.sync_copy(
                x_hbm.at[pl.ds(jt * 8, 8), pl.ds(col, LANE)], ibuf)
            nrows = lax.min(B_J - jt * 8, 8)

            def jr_body(jr, c3):
                u = itl * B_J + jt * 8 + jr
                for c in range(8):
                    v = ibuf[jr, pl.ds(c * 16, 16)]
                    sidx[u, pl.ds(c * 16, 16)] = lax.shift_right_logical(v, 1)
                    par[u, pl.ds(c * 16, 16)] = lax.shift_left(
                        lax.bitwise_and(v, 1), 6)
                return c3

            lax.fori_loop(0, nrows, jr_body, 0)
            return c2

        lax.fori_loop(0, (B_J + 7) // 8, jt_body, 0)

        def fire(u, gb, gsem):
            pltpu.async_copy(tpk_hbm.at[sidx.at[itl * B_J + u]], gb, gsem)

        def proc(u, gb, rb, gsem, ssem):
            @pl.when(u >= 2)
            def _():
                pltpu.make_async_copy(
                    rb, out_hbm.at[0, :, pl.ds(col, LANE)], ssem).wait()
            pltpu.make_async_copy(
                tpk_hbm.at[sidx.at[itl * B_J + u]], gb, gsem).wait()
            row = itl * B_J + u
            for c in range(8):
                pvec = par[row, pl.ds(c * 16, 16)]
                rows_c = iota + (c * 16)

                def d_body(dblk, c4):
                    base = pvec + dblk * 8
                    for dd in range(8):
                        vec = plsc.load_gather(gb, [rows_c, base + dd])
                        rb[dblk * 8 + dd, pl.ds(c * 16, 16)] = vec
                    return c4

                lax.fori_loop(0, EMB_DIM // 8, d_body, 0)
            pltpu.async_copy(rb, out_hbm.at[u, :, pl.ds(col, LANE)], ssem)

        fire(0, gb0, gsem0)
        fire(1, gb1, gsem1)

        def u_body(i, c2):
            u0 = 2 * i
            proc(u0, gb0, rb0, gsem0, ssem0)

            @pl.when(u0 + 2 < B_J)
            def _():
                fire(u0 + 2, gb0, gsem0)
            u1 = 2 * i + 1
            proc(u1, gb1, rb1, gsem1, ssem1)

            @pl.when(u1 + 2 < B_J)
            def _():
                fire(u1 + 2, gb1, gsem1)
            return c2

        lax.fori_loop(0, B_J // 2, u_body, 0)

        pltpu.make_async_copy(
            rb0, out_hbm.at[0, :, pl.ds(col, LANE)], ssem0).wait()
        pltpu.make_async_copy(
            rb1, out_hbm.at[0, :, pl.ds(col, LANE)], ssem1).wait()
        return carry

    lax.fori_loop(0, IT_PER_W, it_body, 0)


def kernel(x, table):
    x_t = x.T.astype(jnp.int32)
    tpk = jnp.reshape(table[:VOCAB], (VOCAB // 2, 128))

    mesh = plsc.VectorSubcoreMesh(core_axis_name="c", subcore_axis_name="s")
    out3 = pl.kernel(
        _emb_body,
        out_type=jax.ShapeDtypeStruct((B_J, EMB_DIM, B_I), jnp.float32),
        mesh=mesh,
        scratch_types=[
            pltpu.VMEM((8, LANE), jnp.int32),
            pltpu.VMEM((UNITS, LANE), jnp.int32),
            pltpu.VMEM((UNITS, LANE), jnp.int32),
            pltpu.VMEM((LANE, LANE), jnp.float32),
            pltpu.VMEM((LANE, LANE), jnp.float32),
            pltpu.VMEM((EMB_DIM, LANE), jnp.float32),
            pltpu.VMEM((EMB_DIM, LANE), jnp.float32),
            pltpu.SemaphoreType.DMA,
            pltpu.SemaphoreType.DMA,
            pltpu.SemaphoreType.DMA,
            pltpu.SemaphoreType.DMA,
        ],
        compiler_params=pltpu.CompilerParams(use_tc_tiling_on_sc=True,
                                             needs_layout_passes=False),
    )(x_t, tpk)
    return out3.transpose(2, 0, 1)


# speedup vs baseline: 1.1796x; 1.1796x over previous
import jax
import jax.numpy as jnp
from jax import lax
from jax.experimental import pallas as pl
from jax.experimental.pallas import tpu as pltpu
from jax.experimental.pallas import tpu_sc as plsc

VOCAB = 1000000
EMB_DIM = 64
NC = 2
NS = 16
NW = NC * NS

CHUNK = 128
B_TOTAL = 16384 * 50
PER_W = B_TOTAL // NW
UNITS = PER_W // CHUNK


def _emb_body(x_hbm, tpk_hbm, out_hbm, xall, sidx, gb0, gb1, rb0, rb1,
              gsem0, gsem1, ssem0, ssem1):
    wid = lax.axis_index("s") * NC + lax.axis_index("c")
    base = wid * PER_W

    pltpu.sync_copy(x_hbm.at[wid], xall)

    def fire(u, sb, gb, gsem):
        for c in range(8):
            v = xall[u, pl.ds(c * 16, 16)]
            sidx[sb, pl.ds(c * 16, 16)] = lax.shift_right_logical(v, 1)
        pltpu.async_copy(tpk_hbm.at[sidx.at[sb]], gb, gsem)

    def proc(u, gb, rb, gsem, ssem):
        @pl.when(u >= 2)
        def _():
            pltpu.make_async_copy(rb, out_hbm.at[pl.ds(0, CHUNK), :],
                                  ssem).wait()
        pltpu.make_async_copy(tpk_hbm.at[sidx.at[0]], gb, gsem).wait()

        def g_body(g, c2):
            pvec = xall[u, pl.ds(g * 16, 16)]
            for r0 in range(16):
                off = lax.shift_left(lax.bitwise_and(pvec[r0], 1), 6)
                r = g * 16 + r0
                for k in range(4):
                    rb[r, pl.ds(k * 16, 16)] = gb[r, pl.ds(off + k * 16, 16)]
            return c2

        lax.fori_loop(0, CHUNK // 16, g_body, 0)
        pltpu.async_copy(rb, out_hbm.at[pl.ds(base + u * CHUNK, CHUNK), :],
                         ssem)

    fire(0, 0, gb0, gsem0)
    fire(1, 1, gb1, gsem1)

    def u_body(i, c2):
        u0 = 2 * i
        proc(u0, gb0, rb0, gsem0, ssem0)

        @pl.when(u0 + 2 < UNITS)
        def _():
            fire(u0 + 2, 0, gb0, gsem0)
        u1 = 2 * i + 1
        proc(u1, gb1, rb1, gsem1, ssem1)

        @pl.when(u1 + 2 < UNITS)
        def _():
            fire(u1 + 2, 1, gb1, gsem1)
        return c2

    lax.fori_loop(0, UNITS // 2, u_body, 0)

    pltpu.make_async_copy(rb0, out_hbm.at[pl.ds(0, CHUNK), :], ssem0).wait()
    pltpu.make_async_copy(rb1, out_hbm.at[pl.ds(0, CHUNK), :], ssem1).wait()


def kernel(x, table):
    x3 = jnp.reshape(x, (NW, UNITS, CHUNK)).astype(jnp.int32)
    tpk = jnp.reshape(table[:VOCAB], (VOCAB // 2, 128))

    mesh = plsc.VectorSubcoreMesh(core_axis_name="c", subcore_axis_name="s")
    out = pl.kernel(
        _emb_body,
        out_type=jax.ShapeDtypeStruct((B_TOTAL, EMB_DIM), jnp.float32),
        mesh=mesh,
        scratch_types=[
            pltpu.VMEM((UNITS, CHUNK), jnp.int32),
            pltpu.VMEM((2, CHUNK), jnp.int32),
            pltpu.VMEM((CHUNK, CHUNK), jnp.float32),
            pltpu.VMEM((CHUNK, CHUNK), jnp.float32),
            pltpu.VMEM((CHUNK, EMB_DIM), jnp.float32),
            pltpu.VMEM((CHUNK, EMB_DIM), jnp.float32),
            pltpu.SemaphoreType.DMA,
            pltpu.SemaphoreType.DMA,
            pltpu.SemaphoreType.DMA,
            pltpu.SemaphoreType.DMA,
        ],
        compiler_params=pltpu.CompilerParams(use_tc_tiling_on_sc=True,
                                             needs_layout_passes=False),
    )(x3, tpk)
    return out.reshape(16384, 50, EMB_DIM)
